# Initial kernel scaffold; baseline (speedup 1.0000x reference)
import jax
import jax.numpy as jnp
from jax.experimental import pallas as pl


def kernel(x, mtlu_y, mtlu_y_shift):
    raise NotImplementedError("write your pallas kernel here")


# speedup vs baseline: 832.2757x; 832.2757x over previous
import dataclasses
import functools

import jax
import jax.numpy as jnp
from jax import lax
from jax.experimental import pallas as pl
from jax.experimental.pallas import tpu as pltpu
from jax.experimental.pallas import tpu_sc as plsc

BIN_NUM = 20
BIN_WIDTH = 0.1
FEAT = 96
HALF = BIN_NUM // 2

NC = 2
NS = 16
L = 16
NW = NC * NS

PLANE = 384 * 384
N_PLANES = 4 * FEAT
PPW = N_PLANES // NW
CHUNKS_PER_PLANE = 8
CHUNK = PLANE // CHUNKS_PER_PLANE
NVEC = CHUNK // L


@jax.jit
def _sc_mtlu(xflat, wflat, bflat):
    mesh = plsc.VectorSubcoreMesh(core_axis_name="c", subcore_axis_name="s")
    cp = pltpu.CompilerParams()
    if "needs_layout_passes" in pltpu.CompilerParams.__dataclass_fields__:
        cp = dataclasses.replace(cp, needs_layout_passes=False)

    @functools.partial(
        pl.kernel,
        compiler_params=cp,
        out_type=jax.ShapeDtypeStruct(xflat.shape, jnp.float32),
        mesh=mesh,
        scratch_types=[
            pltpu.VMEM((FEAT * BIN_NUM,), jnp.float32),
            pltpu.VMEM((FEAT * BIN_NUM,), jnp.float32),
            pltpu.VMEM((CHUNK,), jnp.float32),
            pltpu.VMEM((CHUNK,), jnp.float32),
            pltpu.SemaphoreType.DMA,
        ],
    )
    def k(x_hbm, w_hbm, b_hbm, o_hbm, tw, tb, xin, xout, sem):
        wid = lax.axis_index("s") * NC + lax.axis_index("c")
        pltpu.sync_copy(w_hbm, tw)
        pltpu.sync_copy(b_hbm, tb)

        @pl.loop(0, PPW * CHUNKS_PER_PLANE)
        def _chunk(t):
            plane = wid * PPW + t // CHUNKS_PER_PLANE
            c20 = (plane % FEAT) * BIN_NUM
            off = plane * PLANE + (t % CHUNKS_PER_PLANE) * CHUNK
            pltpu.sync_copy(x_hbm.at[pl.ds(off, CHUNK)], xin)

            @pl.loop(0, NVEC)
            def _vec(i):
                xv = xin[pl.ds(i * L, L)]
                q = xv / BIN_WIDTH
                ti = q.astype(jnp.int32)
                tf = ti.astype(jnp.float32)
                fl = jnp.where(q < tf, ti - 1, ti)
                j = jnp.minimum(jnp.maximum(fl + HALF, 0), BIN_NUM - 1)
                idx = j + c20
                wv = plsc.load_gather(tw, [idx])
                bv = plsc.load_gather(tb, [idx])
                xout[pl.ds(i * L, L)] = wv * xv + bv

            pltpu.sync_copy(xout, o_hbm.at[pl.ds(off, CHUNK)])

    return k(xflat, wflat, bflat)


def kernel(x, mtlu_y, mtlu_y_shift):
    index = jnp.arange(-HALF + 1, HALF + 1, dtype=jnp.float32)
    weight = (mtlu_y - mtlu_y_shift) / BIN_WIDTH
    bias = mtlu_y - (mtlu_y - mtlu_y_shift) * index
    out = _sc_mtlu(x.reshape(-1), weight.reshape(-1), bias.reshape(-1))
    return out.reshape(x.shape)


# speedup vs baseline: 1227.1520x; 1.4745x over previous
import dataclasses
import functools

import jax
import jax.numpy as jnp
from jax import lax
from jax.experimental import pallas as pl
from jax.experimental.pallas import tpu as pltpu
from jax.experimental.pallas import tpu_sc as plsc

BIN_NUM = 20
BIN_WIDTH = 0.1
FEAT = 96
HALF = BIN_NUM // 2

NC = 2
NS = 16
L = 16
NW = NC * NS

PLANE = 384 * 384
N_PLANES = 4 * FEAT
PPW = N_PLANES // NW
CHUNKS_PER_PLANE = 8
CHUNK = PLANE // CHUNKS_PER_PLANE
NVEC = CHUNK // L


@jax.jit
def _sc_mtlu(xflat, wflat, bflat):
    mesh = plsc.VectorSubcoreMesh(core_axis_name="c", subcore_axis_name="s")
    cp = pltpu.CompilerParams()
    if "needs_layout_passes" in pltpu.CompilerParams.__dataclass_fields__:
        cp = dataclasses.replace(cp, needs_layout_passes=False)

    @functools.partial(
        pl.kernel,
        compiler_params=cp,
        out_type=jax.ShapeDtypeStruct(xflat.shape, jnp.float32),
        mesh=mesh,
        scratch_types=[
            pltpu.VMEM((FEAT * BIN_NUM,), jnp.float32),
            pltpu.VMEM((FEAT * BIN_NUM,), jnp.float32),
            pltpu.VMEM((CHUNK,), jnp.float32),
            pltpu.VMEM((CHUNK,), jnp.float32),
            pltpu.SemaphoreType.DMA,
        ],
    )
    def k(x_hbm, w_hbm, b_hbm, o_hbm, tw, tb, xin, xout, sem):
        wid = lax.axis_index("s") * NC + lax.axis_index("c")
        pltpu.sync_copy(w_hbm, tw)
        pltpu.sync_copy(b_hbm, tb)

        @pl.loop(0, PPW * CHUNKS_PER_PLANE)
        def _chunk(t):
            plane = wid * PPW + t // CHUNKS_PER_PLANE
            c20 = (plane % FEAT) * BIN_NUM
            off = plane * PLANE + (t % CHUNKS_PER_PLANE) * CHUNK
            pltpu.sync_copy(x_hbm.at[pl.ds(off, CHUNK)], xin)

            @plsc.parallel_loop(0, CHUNK, step=L, unroll=8)
            def _vec(i):
                xv = xin[pl.ds(i, L)]
                q = xv / BIN_WIDTH
                ti = q.astype(jnp.int32)
                tf = ti.astype(jnp.float32)
                fl = jnp.where(q < tf, ti - 1, ti)
                j = jnp.minimum(jnp.maximum(fl + HALF, 0), BIN_NUM - 1)
                idx = j + c20
                wv = plsc.load_gather(tw, [idx])
                bv = plsc.load_gather(tb, [idx])
                xout[pl.ds(i, L)] = wv * xv + bv

            pltpu.sync_copy(xout, o_hbm.at[pl.ds(off, CHUNK)])

    return k(xflat, wflat, bflat)


def kernel(x, mtlu_y, mtlu_y_shift):
    index = jnp.arange(-HALF + 1, HALF + 1, dtype=jnp.float32)
    weight = (mtlu_y - mtlu_y_shift) / BIN_WIDTH
    bias = mtlu_y - (mtlu_y - mtlu_y_shift) * index
    out = _sc_mtlu(x.reshape(-1), weight.reshape(-1), bias.reshape(-1))
    return out.reshape(x.shape)


# speedup vs baseline: 1574.8965x; 1.2834x over previous
import dataclasses
import functools

import jax
import jax.numpy as jnp
from jax import lax
from jax.experimental import pallas as pl
from jax.experimental.pallas import tpu as pltpu
from jax.experimental.pallas import tpu_sc as plsc

BIN_NUM = 20
BIN_WIDTH = 0.1
FEAT = 96
HALF = BIN_NUM // 2

NC = 2
NS = 16
L = 16
NW = NC * NS

PLANE = 384 * 384
N_PLANES = 4 * FEAT
PPW = N_PLANES // NW
CHUNKS_PER_PLANE = 6
CHUNK = PLANE // CHUNKS_PER_PLANE
NCH = PPW * CHUNKS_PER_PLANE


@jax.jit
def _sc_mtlu(xflat, wflat, bflat):
    mesh = plsc.VectorSubcoreMesh(core_axis_name="c", subcore_axis_name="s")
    cp = pltpu.CompilerParams()
    if "needs_layout_passes" in pltpu.CompilerParams.__dataclass_fields__:
        cp = dataclasses.replace(cp, needs_layout_passes=False)

    @functools.partial(
        pl.kernel,
        compiler_params=cp,
        out_type=jax.ShapeDtypeStruct(xflat.shape, jnp.float32),
        mesh=mesh,
        scratch_types=[
            pltpu.VMEM((FEAT * BIN_NUM,), jnp.float32),
            pltpu.VMEM((FEAT * BIN_NUM,), jnp.float32),
            pltpu.VMEM((CHUNK,), jnp.float32),
            pltpu.VMEM((CHUNK,), jnp.float32),
            pltpu.VMEM((CHUNK,), jnp.float32),
            pltpu.VMEM((CHUNK,), jnp.float32),
            pltpu.SemaphoreType.DMA,
            pltpu.SemaphoreType.DMA,
            pltpu.SemaphoreType.DMA,
            pltpu.SemaphoreType.DMA,
        ],
    )
    def k(x_hbm, w_hbm, b_hbm, o_hbm, tw, tb,
          xin0, xin1, xout0, xout1, si0, si1, so0, so1):
        wid = lax.axis_index("s") * NC + lax.axis_index("c")
        pltpu.sync_copy(w_hbm, tw)
        pltpu.sync_copy(b_hbm, tb)
        xin = (xin0, xin1)
        xout = (xout0, xout1)
        sin = (si0, si1)
        sout = (so0, so1)

        def chunk_off(t):
            plane = wid * PPW + t // CHUNKS_PER_PLANE
            c20 = (plane % FEAT) * BIN_NUM
            off = plane * PLANE + (t % CHUNKS_PER_PLANE) * CHUNK
            return off, c20

        def start_in(t, b):
            off, _ = chunk_off(t)
            pltpu.async_copy(x_hbm.at[pl.ds(off, CHUNK)], xin[b], sin[b])

        start_in(0, 0)

        @pl.loop(0, NCH, step=2)
        def _pair(tt):
            for b in range(2):
                t = tt + b
                off, c20 = chunk_off(t)

                @pl.when(t + 1 < NCH)
                def _prefetch():
                    start_in(t + 1, 1 - b)

                pltpu.make_async_copy(
                    x_hbm.at[pl.ds(off, CHUNK)], xin[b], sin[b]).wait()

                @pl.when(t >= 2)
                def _drain():
                    pltpu.make_async_copy(
                        xout[b], o_hbm.at[pl.ds(off, CHUNK)], sout[b]).wait()

                src = xin[b]
                dst = xout[b]

                @plsc.parallel_loop(0, CHUNK, step=L, unroll=8)
                def _vec(i):
                    xv = src[pl.ds(i, L)]
                    q = xv / BIN_WIDTH
                    ti = q.astype(jnp.int32)
                    tf = ti.astype(jnp.float32)
                    fl = jnp.where(q < tf, ti - 1, ti)
                    idx = jnp.minimum(jnp.maximum(fl + (HALF + c20), c20),
                                      c20 + (BIN_NUM - 1))
                    wv = plsc.load_gather(tw, [idx])
                    bv = plsc.load_gather(tb, [idx])
                    dst[pl.ds(i, L)] = wv * xv + bv

                pltpu.async_copy(xout[b], o_hbm.at[pl.ds(off, CHUNK)], sout[b])

        for b in range(2):
            off, _ = chunk_off(NCH - 2 + b)
            pltpu.make_async_copy(
                xout[b], o_hbm.at[pl.ds(off, CHUNK)], sout[b]).wait()

    return k(xflat, wflat, bflat)


def kernel(x, mtlu_y, mtlu_y_shift):
    index = jnp.arange(-HALF + 1, HALF + 1, dtype=jnp.float32)
    weight = (mtlu_y - mtlu_y_shift) / BIN_WIDTH
    bias = mtlu_y - (mtlu_y - mtlu_y_shift) * index
    out = _sc_mtlu(x.reshape(-1), weight.reshape(-1), bias.reshape(-1))
    return out.reshape(x.shape)


# speedup vs baseline: 1779.2404x; 1.1298x over previous
import dataclasses
import functools

import jax
import jax.numpy as jnp
from jax import lax
from jax.experimental import pallas as pl
from jax.experimental.pallas import tpu as pltpu
from jax.experimental.pallas import tpu_sc as plsc

BIN_NUM = 20
BIN_WIDTH = 0.1
FEAT = 96
HALF = BIN_NUM // 2

NC = 2
NS = 16
L = 16
NW = NC * NS

PLANE = 384 * 384
N_PLANES = 4 * FEAT
PPW = N_PLANES // NW
CHUNKS_PER_PLANE = 6
CHUNK = PLANE // CHUNKS_PER_PLANE
NCH = PPW * CHUNKS_PER_PLANE


@jax.jit
def _sc_mtlu(xflat, wflat, bflat):
    mesh = plsc.VectorSubcoreMesh(core_axis_name="c", subcore_axis_name="s")
    cp = pltpu.CompilerParams()
    if "needs_layout_passes" in pltpu.CompilerParams.__dataclass_fields__:
        cp = dataclasses.replace(cp, needs_layout_passes=False)

    @functools.partial(
        pl.kernel,
        compiler_params=cp,
        out_type=jax.ShapeDtypeStruct(xflat.shape, jnp.float32),
        mesh=mesh,
        scratch_types=[
            pltpu.VMEM((FEAT * BIN_NUM,), jnp.float32),
            pltpu.VMEM((FEAT * BIN_NUM,), jnp.float32),
            pltpu.VMEM((CHUNK,), jnp.float32),
            pltpu.VMEM((CHUNK,), jnp.float32),
            pltpu.VMEM((CHUNK,), jnp.float32),
            pltpu.VMEM((CHUNK,), jnp.float32),
            pltpu.SemaphoreType.DMA,
            pltpu.SemaphoreType.DMA,
            pltpu.SemaphoreType.DMA,
            pltpu.SemaphoreType.DMA,
        ],
    )
    def k(x_hbm, w_hbm, b_hbm, o_hbm, tw, tb,
          xin0, xin1, xout0, xout1, si0, si1, so0, so1):
        wid = lax.axis_index("s") * NC + lax.axis_index("c")
        pltpu.sync_copy(w_hbm, tw)
        pltpu.sync_copy(b_hbm, tb)
        xin = (xin0, xin1)
        xout = (xout0, xout1)
        sin = (si0, si1)
        sout = (so0, so1)

        def chunk_off(t):
            plane = wid * PPW + t // CHUNKS_PER_PLANE
            c20 = (plane % FEAT) * BIN_NUM
            off = plane * PLANE + (t % CHUNKS_PER_PLANE) * CHUNK
            return off, c20

        def start_in(t, b):
            off, _ = chunk_off(t)
            pltpu.async_copy(x_hbm.at[pl.ds(off, CHUNK)], xin[b], sin[b])

        start_in(0, 0)

        @pl.loop(0, NCH, step=2)
        def _pair(tt):
            for b in range(2):
                t = tt + b
                off, c20 = chunk_off(t)

                @pl.when(t + 1 < NCH)
                def _prefetch():
                    start_in(t + 1, 1 - b)

                pltpu.make_async_copy(
                    x_hbm.at[pl.ds(off, CHUNK)], xin[b], sin[b]).wait()

                @pl.when(t >= 2)
                def _drain():
                    pltpu.make_async_copy(
                        xout[b], o_hbm.at[pl.ds(off, CHUNK)], sout[b]).wait()

                src = xin[b]
                dst = xout[b]

                @plsc.parallel_loop(0, CHUNK, step=L, unroll=8)
                def _vec(i):
                    xv = src[pl.ds(i, L)]
                    f = xv * jnp.float32(1.0 / BIN_WIDTH) + jnp.float32(HALF)
                    f = jnp.minimum(jnp.maximum(f, jnp.float32(0.0)),
                                    jnp.float32(BIN_NUM - 0.5))
                    idx = f.astype(jnp.int32) + c20
                    wv = plsc.load_gather(tw, [idx])
                    bv = plsc.load_gather(tb, [idx])
                    dst[pl.ds(i, L)] = wv * xv + bv

                pltpu.async_copy(xout[b], o_hbm.at[pl.ds(off, CHUNK)], sout[b])

        for b in range(2):
            off, _ = chunk_off(NCH - 2 + b)
            pltpu.make_async_copy(
                xout[b], o_hbm.at[pl.ds(off, CHUNK)], sout[b]).wait()

    return k(xflat, wflat, bflat)


def kernel(x, mtlu_y, mtlu_y_shift):
    index = jnp.arange(-HALF + 1, HALF + 1, dtype=jnp.float32)
    weight = (mtlu_y - mtlu_y_shift) / BIN_WIDTH
    bias = mtlu_y - (mtlu_y - mtlu_y_shift) * index
    out = _sc_mtlu(x.reshape(-1), weight.reshape(-1), bias.reshape(-1))
    return out.reshape(x.shape)


# speedup vs baseline: 3866.8280x; 2.1733x over previous
import dataclasses
import functools

import jax
import jax.numpy as jnp
from jax import lax
from jax.experimental import pallas as pl
from jax.experimental.pallas import tpu as pltpu
from jax.experimental.pallas import tpu_sc as plsc

BIN_NUM = 20
BIN_WIDTH = 0.1
FEAT = 96
HALF = BIN_NUM // 2

NC = 2
NS = 16
L = 16
NW = NC * NS

ROWS = 384
COLS = 384
N_PLANES = 4 * FEAT
PPW = N_PLANES // NW
BLK_ROWS = 64
BLKS_PER_PLANE = ROWS // BLK_ROWS
NCH = PPW * BLKS_PER_PLANE
CHUNK = BLK_ROWS * COLS
NCH = PPW * BLKS_PER_PLANE


@jax.jit
def _sc_mtlu(x3, wflat, bflat):
    mesh = plsc.VectorSubcoreMesh(core_axis_name="c", subcore_axis_name="s")
    cp = pltpu.CompilerParams()
    if "needs_layout_passes" in pltpu.CompilerParams.__dataclass_fields__:
        cp = dataclasses.replace(cp, needs_layout_passes=False)

    @functools.partial(
        pl.kernel,
        compiler_params=cp,
        out_type=jax.ShapeDtypeStruct(x3.shape, jnp.float32),
        mesh=mesh,
        scratch_types=[
            pltpu.VMEM((FEAT * BIN_NUM,), jnp.float32),
            pltpu.VMEM((FEAT * BIN_NUM,), jnp.float32),
            pltpu.VMEM((BLK_ROWS, COLS), jnp.float32),
            pltpu.VMEM((BLK_ROWS, COLS), jnp.float32),
            pltpu.VMEM((BLK_ROWS, COLS), jnp.float32),
            pltpu.VMEM((BLK_ROWS, COLS), jnp.float32),
            pltpu.SemaphoreType.DMA,
            pltpu.SemaphoreType.DMA,
            pltpu.SemaphoreType.DMA,
            pltpu.SemaphoreType.DMA,
        ],
    )
    def k(x_hbm, w_hbm, b_hbm, o_hbm, tw, tb,
          xin0, xin1, xout0, xout1, si0, si1, so0, so1):
        wid = lax.axis_index("s") * NC + lax.axis_index("c")
        pltpu.sync_copy(w_hbm, tw)
        pltpu.sync_copy(b_hbm, tb)
        xin = (xin0, xin1)
        xout = (xout0, xout1)
        sin = (si0, si1)
        sout = (so0, so1)

        def block_at(t):
            plane = wid * PPW + t // BLKS_PER_PLANE
            c20 = (plane % FEAT) * BIN_NUM
            r0 = (t % BLKS_PER_PLANE) * BLK_ROWS
            return plane, r0, c20

        def start_in(t, b):
            plane, r0, _ = block_at(t)
            pltpu.async_copy(
                x_hbm.at[plane, pl.ds(r0, BLK_ROWS)], xin[b], sin[b])

        start_in(0, 0)

        @pl.loop(0, NCH, step=2)
        def _pair(tt):
            for b in range(2):
                t = tt + b
                plane, r0, c20 = block_at(t)

                @pl.when(t + 1 < NCH)
                def _prefetch():
                    start_in(t + 1, 1 - b)

                pltpu.make_async_copy(
                    x_hbm.at[plane, pl.ds(r0, BLK_ROWS)], xin[b],
                    sin[b]).wait()

                @pl.when(t >= 2)
                def _drain():
                    pltpu.make_async_copy(
                        xout[b], o_hbm.at[plane, pl.ds(r0, BLK_ROWS)],
                        sout[b]).wait()

                src = xin[b]
                dst = xout[b]

                @pl.loop(0, BLK_ROWS)
                def _row(r):
                    @plsc.parallel_loop(0, COLS, step=L, unroll=8)
                    def _vec(i):
                        xv = src[r, pl.ds(i, L)]
                        f = (xv * jnp.float32(1.0 / BIN_WIDTH)
                             + jnp.float32(HALF))
                        f = jnp.minimum(jnp.maximum(f, jnp.float32(0.0)),
                                        jnp.float32(BIN_NUM - 0.5))
                        idx = f.astype(jnp.int32) + c20
                        wv = plsc.load_gather(tw, [idx])
                        bv = plsc.load_gather(tb, [idx])
                        dst[r, pl.ds(i, L)] = wv * xv + bv

                pltpu.async_copy(
                    xout[b], o_hbm.at[plane, pl.ds(r0, BLK_ROWS)], sout[b])

        for b in range(2):
            plane, r0, _ = block_at(NCH - 2 + b)
            pltpu.make_async_copy(
                xout[b], o_hbm.at[plane, pl.ds(r0, BLK_ROWS)],
                sout[b]).wait()

    return k(x3, wflat, bflat)


def kernel(x, mtlu_y, mtlu_y_shift):
    index = jnp.arange(-HALF + 1, HALF + 1, dtype=jnp.float32)
    weight = (mtlu_y - mtlu_y_shift) / BIN_WIDTH
    bias = mtlu_y - (mtlu_y - mtlu_y_shift) * index
    x3 = x.reshape(N_PLANES, ROWS, COLS)
    out = _sc_mtlu(x3, weight.reshape(-1), bias.reshape(-1))
    return out.reshape(x.shape)
